# baseline (device time: 674969 ns/iter reference)
import functools

import jax
import jax.numpy as jnp
from jax import lax
from jax.experimental import pallas as pl
from jax.experimental.pallas import tpu as pltpu

N_DEV = 4
BC = 512
BLOCKS_PER_DIR = 8
N_PASSES = 2
TOTAL_SENDS = N_PASSES * 6


def _gelu(y):
    c = 0.7978845608028654
    return 0.5 * y * (1.0 + jnp.tanh(c * (y + 0.044715 * (y * y * y))))


def kernel(x, w_mat):
    m, k_local = x.shape
    _, n = w_mat.shape
    assert n == 2 * BLOCKS_PER_DIR * BC
    x = x.astype(jnp.bfloat16)
    w_mat = w_mat.astype(jnp.bfloat16)

    def body(x_ref, w_ref, out_ref, send_bufs, recv_bufs, w_slices,
             send_sems, recv_sems, store_sems, credit_sems, w_sems):
        d = lax.axis_index("i")
        left = lax.rem(d + 3, N_DEV)
        right = lax.rem(d + 1, N_DEV)

        barrier = pltpu.get_barrier_semaphore()
        for nbr in (left, right):
            pl.semaphore_signal(barrier, 1, device_id=(nbr,),
                                device_id_type=pl.DeviceIdType.MESH)
        pl.semaphore_wait(barrier, 2)

        state = [
            dict(di=0, sgn=-1, dst=right, src=left, base=0,
                 n=0, k=0, m=0, f=0, g=0, q=[], fq=[], dot=None, acc0=None,
                 todo=[(p, s) for p in range(N_PASSES) for s in range(4)]),
            dict(di=1, sgn=+1, dst=left, src=right, base=BLOCKS_PER_DIR,
                 n=0, k=0, m=0, f=0, g=0, q=[], fq=[], dot=None, acc0=None,
                 todo=[(p, s) for p in range(N_PASSES) for s in range(4)]),
        ]

        def blk_col(st, p, idx):
            j = lax.rem(d + 2 * N_DEV + st["sgn"] * idx, N_DEV)
            return (st["base"] + p * N_DEV) * BC + j * BC

        def fetch_w(st):
            if not st["todo"]:
                return
            p, s = st["todo"].pop(0)
            fslot = st["f"] % 2
            cp = pltpu.make_async_copy(
                w_ref.at[:, pl.ds(blk_col(st, p, 1 + s), BC)],
                w_slices.at[st["di"], fslot],
                w_sems.at[st["di"], fslot])
            cp.start()
            st["fq"].append(cp)
            st["f"] += 1

        def mm(st):
            st["fq"].pop(0).wait()
            gslot = st["g"] % 2
            st["g"] += 1
            return jnp.dot(x_ref[:, :], w_slices[st["di"], gslot, :, :],
                           preferred_element_type=jnp.float32
                           ).astype(jnp.bfloat16)

        def start_send(st, src_ref):
            nslot = st["n"] % 2
            if st["n"] >= 2:
                pl.semaphore_wait(credit_sems.at[st["di"], nslot], 1)
            rdma = pltpu.make_async_remote_copy(
                src_ref=src_ref,
                dst_ref=recv_bufs.at[st["di"], nslot],
                send_sem=send_sems.at[st["di"]],
                recv_sem=recv_sems.at[st["di"], nslot],
                device_id=(st["dst"],),
                device_id_type=pl.DeviceIdType.MESH,
            )
            rdma.start()
            st["n"] += 1
            return rdma

        def recv_wait(st):
            kslot = st["k"] % 2
            rdma = pltpu.make_async_remote_copy(
                src_ref=recv_bufs.at[st["di"], kslot],
                dst_ref=recv_bufs.at[st["di"], kslot],
                send_sem=send_sems.at[st["di"]],
                recv_sem=recv_sems.at[st["di"], kslot],
                device_id=(st["src"],),
                device_id_type=pl.DeviceIdType.MESH,
            )
            rdma.wait_recv()
            return kslot

        def start_store(st, src_ref, col):
            cp = pltpu.make_async_copy(
                src_ref, out_ref.at[:, pl.ds(col, BC)],
                store_sems.at[st["di"], st["m"] % 2])
            cp.start()
            st["m"] += 1
            return cp

        def queue(st, rdma=None, cp=None, ck=None):
            st["q"].append((rdma, cp, ck))

        def drain(st):
            for rdma, cp, ck in st["q"]:
                if rdma is not None:
                    rdma.wait_send()
                if cp is not None:
                    cp.wait()
                if ck is not None and ck + 2 < TOTAL_SENDS:
                    pl.semaphore_signal(
                        credit_sems.at[st["di"], ck % 2], 1,
                        device_id=(st["src"],),
                        device_id_type=pl.DeviceIdType.MESH)
            st["q"] = []

        for st in state:
            fetch_w(st)
            fetch_w(st)

        for p in range(N_PASSES):
            for st in state:
                acc0 = st["acc0"] if st["acc0"] is not None else mm(st)
                st["acc0"] = None
                drain(st)
                send_bufs[st["di"], :, :] = acc0
                fetch_w(st)
                queue(st, rdma=start_send(st, send_bufs.at[st["di"]]))

            for s in (1, 2, 3):
                for st in state:
                    st["dot"] = mm(st)
                for st in state:
                    kslot = recv_wait(st)
                    acc = st["dot"] + recv_bufs[st["di"], kslot, :, :]
                    drain(st)
                    if s < 3:
                        send_bufs[st["di"], :, :] = acc
                        fetch_w(st)
                        queue(st, rdma=start_send(st, send_bufs.at[st["di"]]),
                              ck=st["k"])
                    else:
                        send_bufs[st["di"], :, :] = _gelu(
                            acc.astype(jnp.float32)).astype(jnp.bfloat16)
                        fetch_w(st)
                        queue(st, rdma=start_send(st, send_bufs.at[st["di"]]),
                              ck=st["k"])
                        own_col = (st["base"] + p * N_DEV) * BC + d * BC
                        queue(st, cp=start_store(
                            st, send_bufs.at[st["di"]], own_col))
                    st["k"] += 1

            for a in range(3):
                for st in state:
                    drain(st)
                    if a == 2 and p + 1 < N_PASSES:
                        st["acc0"] = mm(st)
                for st in state:
                    kslot = recv_wait(st)
                    rdma = None
                    if a < 2:
                        rdma = start_send(st, recv_bufs.at[st["di"], kslot])
                    cp = start_store(st, recv_bufs.at[st["di"], kslot],
                                     blk_col(st, p, 1 + a))
                    queue(st, rdma=rdma, cp=cp, ck=st["k"])
                    st["k"] += 1

        for st in state:
            drain(st)

        @functools.partial(pl.run_scoped,
                           second_barrier=pltpu.SemaphoreType.REGULAR)
        def _(second_barrier):
            for nbr in (left, right):
                pl.semaphore_signal(second_barrier, 1, device_id=(nbr,),
                                    device_id_type=pl.DeviceIdType.MESH)
            pl.semaphore_wait(second_barrier, 2)

    return pl.pallas_call(
        body,
        out_shape=jax.ShapeDtypeStruct((m, n), jnp.bfloat16),
        in_specs=[
            pl.BlockSpec(memory_space=pltpu.MemorySpace.VMEM),
            pl.BlockSpec(memory_space=pl.ANY),
        ],
        out_specs=pl.BlockSpec(memory_space=pl.ANY),
        scratch_shapes=[
            pltpu.VMEM((2, m, BC), jnp.bfloat16),
            pltpu.VMEM((2, 2, m, BC), jnp.bfloat16),
            pltpu.VMEM((2, 2, k_local, BC), jnp.bfloat16),
            pltpu.SemaphoreType.DMA((2,)),
            pltpu.SemaphoreType.DMA((2, 2)),
            pltpu.SemaphoreType.DMA((2, 2)),
            pltpu.SemaphoreType.REGULAR((2, 2)),
            pltpu.SemaphoreType.DMA((2, 2)),
        ],
        compiler_params=pltpu.CompilerParams(
            collective_id=0,
            vmem_limit_bytes=64 * 1024 * 1024,
            has_side_effects=True,
        ),
    )(x, w_mat)


# device time: 661156 ns/iter; 1.0209x vs baseline; 1.0209x over previous
import functools

import jax
import jax.numpy as jnp
from jax import lax
from jax.experimental import pallas as pl
from jax.experimental.pallas import tpu as pltpu

N_DEV = 4
BC = 512
HC = BC // 2
BLOCKS_PER_DIR = 8
N_PASSES = 2
TOTAL_SENDS = N_PASSES * 6


def _gelu(y):
    c = 0.7978845608028654
    return 0.5 * y * (1.0 + jnp.tanh(c * (y + 0.044715 * (y * y * y))))


def kernel(x, w_mat):
    m, k_local = x.shape
    _, n = w_mat.shape
    assert n == 2 * BLOCKS_PER_DIR * BC
    x = x.astype(jnp.bfloat16)
    w_mat = w_mat.astype(jnp.bfloat16)

    def body(x_ref, w_ref, out_ref, send_bufs, recv_bufs, w_slices,
             send_sems, recv_sems, store_sems, credit_sems, w_sems):
        d = lax.axis_index("i")
        left = lax.rem(d + 3, N_DEV)
        right = lax.rem(d + 1, N_DEV)

        barrier = pltpu.get_barrier_semaphore()
        for nbr in (left, right):
            pl.semaphore_signal(barrier, 1, device_id=(nbr,),
                                device_id_type=pl.DeviceIdType.MESH)
        pl.semaphore_wait(barrier, 2)

        state = [
            dict(di=0, sgn=-1, dst=right, src=left, base=0,
                 n=0, k=0, m=0, f=0, g=0, q=[], fq=[], dot=None, acc0=None,
                 todo=[(p, s) for p in range(N_PASSES) for s in range(4)]),
            dict(di=1, sgn=+1, dst=left, src=right, base=BLOCKS_PER_DIR,
                 n=0, k=0, m=0, f=0, g=0, q=[], fq=[], dot=None, acc0=None,
                 todo=[(p, s) for p in range(N_PASSES) for s in range(4)]),
        ]

        def blk_col(st, p, idx):
            j = lax.rem(d + 2 * N_DEV + st["sgn"] * idx, N_DEV)
            return (st["base"] + p * N_DEV) * BC + j * BC

        def fetch_w(st):
            if not st["todo"]:
                return
            p, s = st["todo"].pop(0)
            fslot = st["f"] % 2
            cp = pltpu.make_async_copy(
                w_ref.at[:, pl.ds(blk_col(st, p, 1 + s), BC)],
                w_slices.at[st["di"], fslot],
                w_sems.at[st["di"], fslot])
            cp.start()
            st["fq"].append(cp)
            st["f"] += 1

        def mm(st):
            st["fq"].pop(0).wait()
            gslot = st["g"] % 2
            st["g"] += 1
            return jnp.dot(x_ref[:, :], w_slices[st["di"], gslot, :, :],
                           preferred_element_type=jnp.float32
                           ).astype(jnp.bfloat16)

        def start_send_chunk(st, src_ref, c):
            nslot = st["n"] % 2
            if c == 0 and st["n"] >= 2:
                pl.semaphore_wait(credit_sems.at[st["di"], nslot], 1)
            rdma = pltpu.make_async_remote_copy(
                src_ref=src_ref,
                dst_ref=recv_bufs.at[st["di"], nslot, :,
                                     pl.ds(c * HC, HC)],
                send_sem=send_sems.at[st["di"], c],
                recv_sem=recv_sems.at[st["di"], nslot, c],
                device_id=(st["dst"],),
                device_id_type=pl.DeviceIdType.MESH,
            )
            rdma.start()
            if c == 1:
                st["n"] += 1
            return rdma

        def recv_wait_chunk(st, c):
            kslot = st["k"] % 2
            rdma = pltpu.make_async_remote_copy(
                src_ref=recv_bufs.at[st["di"], kslot, :,
                                     pl.ds(c * HC, HC)],
                dst_ref=recv_bufs.at[st["di"], kslot, :,
                                     pl.ds(c * HC, HC)],
                send_sem=send_sems.at[st["di"], c],
                recv_sem=recv_sems.at[st["di"], kslot, c],
                device_id=(st["src"],),
                device_id_type=pl.DeviceIdType.MESH,
            )
            rdma.wait_recv()
            return kslot

        def start_store(st, src_ref, col):
            cp = pltpu.make_async_copy(
                src_ref, out_ref.at[:, pl.ds(col, BC)],
                store_sems.at[st["di"], st["m"] % 2])
            cp.start()
            st["m"] += 1
            return cp

        def queue(st, rdma=None, cp=None, ck=None):
            st["q"].append((rdma, cp, ck))

        def drain(st):
            for rdma, cp, ck in st["q"]:
                if rdma is not None:
                    rdma.wait_send()
                if cp is not None:
                    cp.wait()
                if ck is not None and ck + 2 < TOTAL_SENDS:
                    pl.semaphore_signal(
                        credit_sems.at[st["di"], ck % 2], 1,
                        device_id=(st["src"],),
                        device_id_type=pl.DeviceIdType.MESH)
            st["q"] = []

        for st in state:
            fetch_w(st)
            fetch_w(st)

        for p in range(N_PASSES):
            for st in state:
                acc0 = st["acc0"] if st["acc0"] is not None else mm(st)
                st["acc0"] = None
                drain(st)
                send_bufs[st["di"], :, :] = acc0
                fetch_w(st)
                for c in (0, 1):
                    queue(st, rdma=start_send_chunk(
                        st, send_bufs.at[st["di"], :, pl.ds(c * HC, HC)], c))

            for s in (1, 2, 3):
                for st in state:
                    st["dot"] = mm(st)
                for st in state:
                    drain(st)
                    for c in (0, 1):
                        kslot = recv_wait_chunk(st, c)
                        cl, ch = c * HC, (c + 1) * HC
                        acc = (st["dot"][:, cl:ch]
                               + recv_bufs[st["di"], kslot, :, cl:ch])
                        if s == 3:
                            acc = _gelu(acc.astype(jnp.float32)).astype(
                                jnp.bfloat16)
                        send_bufs[st["di"], :, cl:ch] = acc
                        queue(st, rdma=start_send_chunk(
                            st, send_bufs.at[st["di"], :, pl.ds(cl, HC)], c),
                            ck=st["k"] if c == 1 else None)
                    fetch_w(st)
                    if s == 3:
                        own_col = (st["base"] + p * N_DEV) * BC + d * BC
                        queue(st, cp=start_store(
                            st, send_bufs.at[st["di"]], own_col))
                    st["k"] += 1

            for a in range(3):
                for st in state:
                    drain(st)
                    if a == 2 and p + 1 < N_PASSES:
                        st["acc0"] = mm(st)
                for st in state:
                    kslot = st["k"] % 2
                    for c in (0, 1):
                        recv_wait_chunk(st, c)
                        if a < 2:
                            queue(st, rdma=start_send_chunk(
                                st, recv_bufs.at[st["di"], kslot, :,
                                                 pl.ds(c * HC, HC)], c))
                    cp = start_store(st, recv_bufs.at[st["di"], kslot],
                                     blk_col(st, p, 1 + a))
                    queue(st, cp=cp, ck=st["k"])
                    st["k"] += 1

        for st in state:
            drain(st)

        @functools.partial(pl.run_scoped,
                           second_barrier=pltpu.SemaphoreType.REGULAR)
        def _(second_barrier):
            for nbr in (left, right):
                pl.semaphore_signal(second_barrier, 1, device_id=(nbr,),
                                    device_id_type=pl.DeviceIdType.MESH)
            pl.semaphore_wait(second_barrier, 2)

    return pl.pallas_call(
        body,
        out_shape=jax.ShapeDtypeStruct((m, n), jnp.bfloat16),
        in_specs=[
            pl.BlockSpec(memory_space=pltpu.MemorySpace.VMEM),
            pl.BlockSpec(memory_space=pl.ANY),
        ],
        out_specs=pl.BlockSpec(memory_space=pl.ANY),
        scratch_shapes=[
            pltpu.VMEM((2, m, BC), jnp.bfloat16),
            pltpu.VMEM((2, 2, m, BC), jnp.bfloat16),
            pltpu.VMEM((2, 2, k_local, BC), jnp.bfloat16),
            pltpu.SemaphoreType.DMA((2, 2)),
            pltpu.SemaphoreType.DMA((2, 2, 2)),
            pltpu.SemaphoreType.DMA((2, 2)),
            pltpu.SemaphoreType.REGULAR((2, 2)),
            pltpu.SemaphoreType.DMA((2, 2)),
        ],
        compiler_params=pltpu.CompilerParams(
            collective_id=0,
            vmem_limit_bytes=64 * 1024 * 1024,
            has_side_effects=True,
        ),
    )(x, w_mat)
